# baseline (speedup 1.0000x reference)
import functools

import jax
import jax.numpy as jnp
from jax import lax
from jax.experimental import pallas as pl
from jax.experimental.pallas import tpu as pltpu
from jax.experimental.pallas import tpu_sc as plsc

B = 4096
NEG = 200
D = 32
USER_NUM = 100000
ITEM_NUM = 1000000
WEIGHT = 0.5
MARGIN1 = 4.0
MARGIN2 = 0.5
GAMMA = 1e-4

NC = 2
NS = 16
NW = NC * NS
UPW = B // NW
CH_USERS = 4
CH_ROWS = CH_USERS * NEG
NCH = UPW // CH_USERS

IT_PW = 31248
IT_TC = 496
IT_TNC = IT_PW // IT_TC
IT_TAIL_BASE = IT_PW * NW
IT_TAIL = ITEM_NUM - IT_TAIL_BASE
UT_PW = 3120
UT_TC = 240
UT_TNC = UT_PW // UT_TC
UT_TAIL_BASE = UT_PW * NW
UT_TAIL = USER_NUM - UT_TAIL_BASE

_f32 = jnp.float32
_i32 = jnp.int32


def _k1_body(tTi_h, tTu_h, itab_o, utab_o, ss_o,
             inb_a, inb_b, outb_a, outb_b, part_v,
             gin_a, gin_b, gout_a, gout_b, sem):
    wid = lax.axis_index("s") * NC + lax.axis_index("c")
    lane = lax.iota(_i32, 16)
    perm = [(lane + k) & 15 for k in range(16)]
    h16l = [lane, lane + 16]

    def transpose_chunk(inb, outb, n, acc):
        def bb(b, acc):
            iblk = b * 16
            for h in range(2):
                for k in range(16):
                    idx_i = iblk + perm[k]
                    v = plsc.load_gather(inb, [h16l[h], idx_i])
                    plsc.store_scatter(outb, [idx_i, h16l[h]], v)
                    acc = acc + v * v
            return acc

        return lax.fori_loop(0, n // 16, bb, acc)

    def start_in(tab_h, i0, n, inb, gsem):
        pltpu.make_async_copy(tab_h.at[:, pl.ds(i0, n)],
                              inb.at[:, pl.ds(0, n)], gsem).start()

    def wait_in(tab_h, n, inb, gsem):
        pltpu.make_async_copy(tab_h.at[:, pl.ds(0, n)],
                              inb.at[:, pl.ds(0, n)], gsem).wait()

    def start_out(out_h, i0, n, outb, osem):
        pltpu.make_async_copy(outb.at[pl.ds(0, n)],
                              out_h.at[pl.ds(i0, n)], osem).start()

    def wait_out(out_h, n, outb, osem):
        pltpu.make_async_copy(outb.at[pl.ds(0, n)],
                              out_h.at[pl.ds(0, n)], osem).wait()

    def phase(tab_h, out_h, pw, n, nch, acc):
        i0w = wid * pw
        npairs = nch // 2
        start_in(tab_h, i0w, n, inb_a, gin_a)

        def pb(t, acc):
            c0 = 2 * t
            start_in(tab_h, i0w + (c0 + 1) * n, n, inb_b, gin_b)
            wait_in(tab_h, n, inb_a, gin_a)

            @pl.when(t > 0)
            def _():
                wait_out(out_h, n, outb_a, gout_a)

            acc = transpose_chunk(inb_a, outb_a, n, acc)
            start_out(out_h, i0w + c0 * n, n, outb_a, gout_a)

            @pl.when(c0 + 2 < nch)
            def _():
                start_in(tab_h, i0w + (c0 + 2) * n, n, inb_a, gin_a)

            wait_in(tab_h, n, inb_b, gin_b)

            @pl.when(t > 0)
            def _():
                wait_out(out_h, n, outb_b, gout_b)

            acc = transpose_chunk(inb_b, outb_b, n, acc)
            start_out(out_h, i0w + (c0 + 1) * n, n, outb_b, gout_b)
            return acc

        acc = lax.fori_loop(0, npairs, pb, acc)
        if nch % 2 == 1:
            cf = nch - 1
            wait_in(tab_h, n, inb_a, gin_a)
            wait_out(out_h, n, outb_a, gout_a)
            acc = transpose_chunk(inb_a, outb_a, n, acc)
            start_out(out_h, i0w + cf * n, n, outb_a, gout_a)
        wait_out(out_h, n, outb_a, gout_a)
        wait_out(out_h, n, outb_b, gout_b)
        return acc

    acc = jnp.zeros((16,), _f32)
    acc = phase(tTi_h, itab_o, IT_PW, IT_TC, IT_TNC, acc)
    acc = phase(tTu_h, utab_o, UT_PW, UT_TC, UT_TNC, acc)

    def tail(tab_h, out_h, base, n, acc, owner):
        pltpu.sync_copy(tab_h.at[:, pl.ds(base, n)], inb_a.at[:, pl.ds(0, n)])
        t_acc = transpose_chunk(inb_a, outb_a, n, jnp.zeros((16,), _f32))

        @pl.when(wid == owner)
        def _():
            pltpu.sync_copy(outb_a.at[pl.ds(0, n)], out_h.at[pl.ds(base, n)])

        return acc + jnp.where(wid == owner, t_acc, jnp.zeros((16,), _f32))

    acc = tail(tTi_h, itab_o, IT_TAIL_BASE, IT_TAIL, acc, 0)
    acc = tail(tTu_h, utab_o, UT_TAIL_BASE, UT_TAIL, acc, 1)
    part_v[pl.ds(0, 16)] = acc
    pltpu.sync_copy(part_v, ss_o.at[wid])


def _k1_part(tTi, tTu):
    mesh = plsc.VectorSubcoreMesh(core_axis_name="c", subcore_axis_name="s",
                                  num_cores=NC, num_subcores=NS)
    kern = pl.kernel(
        _k1_body,
        out_type=(
            jax.ShapeDtypeStruct((ITEM_NUM, D), _f32),
            jax.ShapeDtypeStruct((USER_NUM, D), _f32),
            jax.ShapeDtypeStruct((NW, 16), _f32),
        ),
        mesh=mesh,
        compiler_params=pltpu.CompilerParams(needs_layout_passes=False,
                                             use_tc_tiling_on_sc=False),
        scratch_types=[
            pltpu.VMEM((D, IT_TC), _f32),
            pltpu.VMEM((D, IT_TC), _f32),
            pltpu.VMEM((IT_TC, D), _f32),
            pltpu.VMEM((IT_TC, D), _f32),
            pltpu.VMEM((16,), _f32),
            pltpu.SemaphoreType.DMA,
            pltpu.SemaphoreType.DMA,
            pltpu.SemaphoreType.DMA,
            pltpu.SemaphoreType.DMA,
            pltpu.SemaphoreType.DMA,
        ],
    )
    return kern(tTi, tTu)


def _k2_body(users_h, pos_h, negf_h, utab_h, itab_h, udeg_h, ideg_h,
             urows_o, prows_o, du_o, sq_o, udeg_o, pdeg_o,
             uidx_v, pidx_v, urows_v, prows_v, udeg_v, pdeg_v, nidx_v,
             nrows_a, nrows_b, dust_a, dust_b, sqst_a, sqst_b,
             gsem_a, gsem_b, osem_a, osem_b, sem):
    wid = lax.axis_index("s") * NC + lax.axis_index("c")
    ubase = wid * UPW
    nbase = ubase * NEG

    pltpu.sync_copy(users_h.at[pl.ds(ubase, UPW)], uidx_v)
    pltpu.sync_copy(pos_h.at[pl.ds(ubase, UPW)], pidx_v)
    pltpu.sync_copy(negf_h.at[pl.ds(nbase, UPW * NEG)], nidx_v)
    pltpu.async_copy(utab_h.at[uidx_v], urows_v, sem).wait()
    pltpu.async_copy(itab_h.at[pidx_v], prows_v, sem).wait()
    pltpu.async_copy(udeg_h.at[uidx_v], udeg_v, sem).wait()
    pltpu.async_copy(ideg_h.at[pidx_v], pdeg_v, sem).wait()
    pltpu.sync_copy(urows_v, urows_o.at[pl.ds(ubase, UPW)])
    pltpu.sync_copy(prows_v, prows_o.at[pl.ds(ubase, UPW)])
    pltpu.sync_copy(udeg_v, udeg_o.at[pl.ds(ubase, UPW)])
    pltpu.sync_copy(pdeg_v, pdeg_o.at[pl.ds(ubase, UPW)])

    lane = lax.iota(_i32, 16)

    def start_gather(c, nrows_v, gsem):
        pltpu.make_async_copy(
            itab_h.at[nidx_v.at[pl.ds(c * CH_ROWS, CH_ROWS)]],
            nrows_v, gsem).start()

    def wait_gather(nrows_v, gsem):
        pltpu.make_async_copy(
            itab_h.at[nidx_v.at[pl.ds(0, CH_ROWS)]], nrows_v, gsem).wait()

    def start_out(c, dust_v, sqst_v, osem):
        off = nbase + c * CH_ROWS
        pltpu.make_async_copy(dust_v, du_o.at[pl.ds(off, CH_ROWS)],
                              osem).start()
        pltpu.make_async_copy(sqst_v, sq_o.at[pl.ds(off, CH_ROWS)],
                              osem).start()

    def wait_out(dust_v, sqst_v, osem):
        pltpu.make_async_copy(dust_v, du_o.at[pl.ds(nbase, CH_ROWS)],
                              osem).wait()
        pltpu.make_async_copy(sqst_v, sq_o.at[pl.ds(nbase, CH_ROWS)],
                              osem).wait()

    def _perm(v, idx):
        return lax.gather(
            v, idx[:, None],
            lax.GatherDimensionNumbers(offset_dims=(),
                                       collapsed_slice_dims=(0,),
                                       start_index_map=(0,)),
            (1,), mode=lax.GatherScatterMode.PROMISE_IN_BOUNDS)

    rot_m = [(lane + d0) & 31 for d0 in range(D)]
    rot_low = [m & 15 for m in rot_m]
    rot_hi = [m >= 16 for m in rot_m]

    def compute_chunk(c, nrows_v, dust_v, sqst_v):
        def user_body(j, carry):
            urow = c * CH_USERS + j
            u0 = urows_v[urow, pl.ds(0, 16)]
            u1 = urows_v[urow, pl.ds(16, 16)]
            uvec = [jnp.where(rot_hi[d0], _perm(u1, rot_low[d0]),
                              _perm(u0, rot_low[d0])) for d0 in range(D)]
            jb = j * NEG

            def grp_body(g, carry):
                gb = jnp.minimum(g * 16, NEG - 16)
                rb = jb + gb
                rowbase = (rb + lane) * D
                accd = jnp.zeros((16,), _f32)
                accq = jnp.zeros((16,), _f32)
                zero16 = jnp.zeros((16,), _i32)
                for d0 in range(D):
                    col = plsc.load_gather(
                        nrows_v, [zero16, rowbase + rot_m[d0]])
                    accd = accd + uvec[d0] * col
                    accq = accq + col * col
                dust_v[pl.ds(rb, 16)] = accd
                sqst_v[pl.ds(rb, 16)] = accq
                return carry

            return lax.fori_loop(0, 13, grp_body, carry)

        lax.fori_loop(0, CH_USERS, user_body, 0)

    start_gather(0, nrows_a, gsem_a)

    def pair_body(t, carry):
        c0 = 2 * t
        c1 = c0 + 1
        start_gather(c1, nrows_b, gsem_b)
        wait_gather(nrows_a, gsem_a)

        @pl.when(t > 0)
        def _():
            wait_out(dust_a, sqst_a, osem_a)

        compute_chunk(c0, nrows_a, dust_a, sqst_a)
        start_out(c0, dust_a, sqst_a, osem_a)

        @pl.when(t < NCH // 2 - 1)
        def _():
            start_gather(c0 + 2, nrows_a, gsem_a)

        wait_gather(nrows_b, gsem_b)

        @pl.when(t > 0)
        def _():
            wait_out(dust_b, sqst_b, osem_b)

        compute_chunk(c1, nrows_b, dust_b, sqst_b)
        start_out(c1, dust_b, sqst_b, osem_b)
        return carry

    lax.fori_loop(0, NCH // 2, pair_body, 0)
    wait_out(dust_a, sqst_a, osem_a)
    wait_out(dust_b, sqst_b, osem_b)


def _k2_part(users, pos, negf, utabL, itabL, user_degree, item_degree):
    mesh = plsc.VectorSubcoreMesh(core_axis_name="c", subcore_axis_name="s",
                                  num_cores=NC, num_subcores=NS)
    kern = pl.kernel(
        _k2_body,
        out_type=(
            jax.ShapeDtypeStruct((B, D), _f32),
            jax.ShapeDtypeStruct((B, D), _f32),
            jax.ShapeDtypeStruct((B * NEG,), _f32),
            jax.ShapeDtypeStruct((B * NEG,), _f32),
            jax.ShapeDtypeStruct((B,), _f32),
            jax.ShapeDtypeStruct((B,), _f32),
        ),
        mesh=mesh,
        compiler_params=pltpu.CompilerParams(needs_layout_passes=False,
                                             use_tc_tiling_on_sc=False),
        scratch_types=[
            pltpu.VMEM((UPW,), _i32),
            pltpu.VMEM((UPW,), _i32),
            pltpu.VMEM((UPW, D), _f32),
            pltpu.VMEM((UPW, D), _f32),
            pltpu.VMEM((UPW,), _f32),
            pltpu.VMEM((UPW,), _f32),
            pltpu.VMEM((UPW * NEG,), _i32),
            pltpu.VMEM((CH_ROWS, D), _f32),
            pltpu.VMEM((CH_ROWS, D), _f32),
            pltpu.VMEM((CH_ROWS,), _f32),
            pltpu.VMEM((CH_ROWS,), _f32),
            pltpu.VMEM((CH_ROWS,), _f32),
            pltpu.VMEM((CH_ROWS,), _f32),
            pltpu.SemaphoreType.DMA,
            pltpu.SemaphoreType.DMA,
            pltpu.SemaphoreType.DMA,
            pltpu.SemaphoreType.DMA,
            pltpu.SemaphoreType.DMA,
        ],
    )
    return kern(users, pos, negf, utabL, itabL, user_degree, item_degree)


RED_GRID = 125


def _min_body(id_ref, ud_ref, mnu_ref, mni_ref):
    i = pl.program_id(0)
    mi = jnp.min(id_ref[...])
    mu = jnp.min(ud_ref[...])

    @pl.when(i == 0)
    def _():
        mni_ref[0, 0] = mi
        mnu_ref[0, 0] = mu

    @pl.when(i > 0)
    def _():
        mni_ref[0, 0] = jnp.minimum(mni_ref[0, 0], mi)
        mnu_ref[0, 0] = jnp.minimum(mnu_ref[0, 0], mu)


def _tc_minred(user_degree, item_degree):
    id2 = item_degree.reshape(RED_GRID, 1, ITEM_NUM // RED_GRID)
    ud2 = user_degree.reshape(RED_GRID, 1, USER_NUM // RED_GRID)
    return pl.pallas_call(
        _min_body,
        grid=(RED_GRID,),
        in_specs=[
            pl.BlockSpec((1, 1, ITEM_NUM // RED_GRID), lambda i: (i, 0, 0)),
            pl.BlockSpec((1, 1, USER_NUM // RED_GRID), lambda i: (i, 0, 0)),
        ],
        out_specs=[
            pl.BlockSpec((1, 1), lambda i: (0, 0), memory_space=pltpu.SMEM),
            pl.BlockSpec((1, 1), lambda i: (0, 0), memory_space=pltpu.SMEM),
        ],
        out_shape=[jax.ShapeDtypeStruct((1, 1), _f32)] * 2,
    )(id2, ud2)


def _loss_body(du_ref, sq_ref, u_ref, p_ref, ud_ref, pd_ref, ss_ref,
               mnu_ref, mni_ref, tot_ref, l1_ref, l2_ref, reg_ref):
    u = u_ref[...]
    p = p_ref[...]
    squ = jnp.sum(u * u, axis=1, keepdims=True)
    dup = jnp.sum(u * p, axis=1, keepdims=True)
    sqp = jnp.sum(p * p, axis=1, keepdims=True)
    cu = jnp.maximum(jnp.sqrt(squ), 1e-12)
    cp = jnp.maximum(jnp.sqrt(sqp), 1e-12)
    du = du_ref[...]
    sq = sq_ref[...]
    cn = jnp.maximum(jnp.sqrt(sq), 1e-12)
    pos_sc = dup / cu
    npos_sc = dup / (cu * cp)
    neg_sc = du / cu
    nneg_sc = du / (cu * cn)
    udeg = ud_ref[...]
    pdeg = pd_ref[...]
    upw = jnp.log(udeg * 1000.0)
    pw = jnp.log(pdeg * 1000.0)
    minu = mnu_ref[0, 0]
    mini = mni_ref[0, 0]
    npw = pw / (-jnp.log(mini * 1000.0 + 1e-7))
    nuw = upw / (-jnp.log(minu * 1000.0 + 1e-7))

    ep1 = jnp.exp((pos_sc + pw + upw) / MARGIN1)
    en1 = jnp.exp(neg_sc / MARGIN1)
    ns1 = jnp.mean(en1, axis=1, keepdims=True)
    d1 = NEG * ns1 + ep1 + 1e-7
    l1 = -jnp.mean(jnp.log(ep1 / d1))

    ep2 = jnp.exp((npos_sc + npw + nuw) / MARGIN2)
    en2 = jnp.exp(nneg_sc / MARGIN2)
    ns2 = jnp.mean(en2, axis=1, keepdims=True)
    d2 = NEG * ns2 + ep2 + 1e-7
    l2 = -jnp.mean(jnp.log(ep2 / d2))

    reg = GAMMA * jnp.sum(ss_ref[...]) / 2.0
    l1w = WEIGHT * l1
    tot_ref[0, 0] = l1w + l2 + reg
    l1_ref[0, 0] = l1w
    l2_ref[0, 0] = l2
    reg_ref[0, 0] = reg


def kernel(users, pos_items, neg_items, user_table, item_table, user_degree,
           item_degree):
    users = users.astype(_i32)
    pos = pos_items.astype(_i32)
    negf = neg_items.astype(_i32).reshape(-1)
    itabL, utabL, ss = _k1_part(item_table.T, user_table.T)
    (urows, prows, du, sq, udeg, pdeg) = _k2_part(
        users, pos, negf, utabL, itabL, user_degree, item_degree)
    mnu, mni = _tc_minred(user_degree, item_degree)
    du2 = du.reshape(B, NEG)
    sq2 = sq.reshape(B, NEG)
    out = pl.pallas_call(
        _loss_body,
        in_specs=[
            pl.BlockSpec((B, NEG), lambda: (0, 0)),
            pl.BlockSpec((B, NEG), lambda: (0, 0)),
            pl.BlockSpec((B, D), lambda: (0, 0)),
            pl.BlockSpec((B, D), lambda: (0, 0)),
            pl.BlockSpec((B, 1), lambda: (0, 0)),
            pl.BlockSpec((B, 1), lambda: (0, 0)),
            pl.BlockSpec((NW, 16), lambda: (0, 0)),
            pl.BlockSpec((1, 1), lambda: (0, 0), memory_space=pltpu.SMEM),
            pl.BlockSpec((1, 1), lambda: (0, 0), memory_space=pltpu.SMEM),
        ],
        out_shape=[jax.ShapeDtypeStruct((1, 1), _f32)] * 4,
        out_specs=[pl.BlockSpec((1, 1), lambda: (0, 0),
                                memory_space=pltpu.SMEM)] * 4,
    )(du2, sq2, urows, prows, udeg.reshape(B, 1), pdeg.reshape(B, 1),
      ss, mnu, mni)
    tot, l1w, l2, reg = out
    return (tot[0, 0], l1w[0, 0], l2[0, 0], reg[0, 0])
